# baseline (speedup 1.0000x reference)
import jax
import jax.numpy as jnp
from jax import lax
from jax.experimental import pallas as pl
from jax.experimental.pallas import tpu as pltpu
from jax.experimental.pallas import tpu_sc as plsc

D_MODEL = 64
BATCH = 16384
SEQ_LEN = 200
B_TOTAL = BATCH * SEQ_LEN

_NC = 2
_NS = 16
_NW = _NC * _NS

_B_PER_W = B_TOTAL // _NW
_CHUNK = 320
_N_CHUNKS = _B_PER_W // _CHUNK
_R = 4
_N_OUTER = _N_CHUNKS // _R


def _gather_kernel(pe_hbm, idx_hbm, out_hbm,
                   table_sh, idx_v, rows_v, sem_i, sem_g, sem_w):
    sid = lax.axis_index("s")
    wid = sid * _NC + lax.axis_index("c")
    base0 = wid * _B_PER_W

    @pl.when(sid == 0)
    def _():
        pltpu.sync_copy(pe_hbm, table_sh)

    plsc.subcore_barrier()

    def fire_idx(g, r):
        off = pl.multiple_of(base0 + g * _CHUNK, _CHUNK)
        pltpu.async_copy(idx_hbm.at[pl.ds(off, _CHUNK)], idx_v.at[r],
                         sem_i.at[r])

    def drain_idx(r):
        pltpu.make_async_copy(idx_hbm.at[pl.ds(0, _CHUNK)], idx_v.at[r],
                              sem_i.at[r]).wait()

    def fire_gather(g, r):
        pltpu.async_copy(table_sh.at[idx_v.at[r]], rows_v.at[r], sem_g.at[r])

    def drain_gather(r):
        pltpu.make_async_copy(pe_hbm.at[pl.ds(0, _CHUNK)], rows_v.at[r],
                              sem_g.at[r]).wait()

    def fire_write(g, r):
        base = pl.multiple_of(base0 + g * _CHUNK, _CHUNK)
        pltpu.async_copy(rows_v.at[r], out_hbm.at[pl.ds(base, _CHUNK)],
                         sem_w.at[r])

    def drain_write(r):
        pltpu.make_async_copy(pe_hbm.at[pl.ds(0, _CHUNK)], rows_v.at[r],
                              sem_w.at[r]).wait()

    fire_idx(0, 0)

    def body(t, _):
        g_base = t * _R
        for r in range(_R):
            g = g_base + r
            r_next = (r + 1) % _R
            r_prev = (r - 1) % _R

            if r == _R - 1:
                drain_write(r_next)

                @pl.when(t < _N_OUTER - 1)
                def _():
                    fire_idx(g + 1, r_next)
            else:

                @pl.when(t > 0)
                def _():
                    drain_write(r_next)

                fire_idx(g + 1, r_next)

            drain_idx(r)
            fire_gather(g, r)

            if r == 0:

                @pl.when(t > 0)
                def _():
                    drain_gather(r_prev)
                    fire_write(g - 1, r_prev)
            else:
                drain_gather(r_prev)
                fire_write(g - 1, r_prev)
        return ()

    lax.fori_loop(0, _N_OUTER, body, (), unroll=False)

    drain_gather(_R - 1)
    fire_write(_N_CHUNKS - 1, _R - 1)
    for r in range(1, _R):
        drain_write(r)


@jax.jit
def _temporal_embedding(positions, pe):
    idx1d = positions.reshape(B_TOTAL)
    mesh = plsc.VectorSubcoreMesh(core_axis_name="c", subcore_axis_name="s")
    out = pl.kernel(
        _gather_kernel,
        out_type=jax.ShapeDtypeStruct((B_TOTAL, D_MODEL), jnp.float32),
        mesh=mesh,
        scratch_types=[
            pltpu.VMEM_SHARED((10000, D_MODEL), jnp.float32),
            pltpu.VMEM((_R, _CHUNK), jnp.int32),
            pltpu.VMEM((_R, _CHUNK, D_MODEL), jnp.float32),
            pltpu.SemaphoreType.DMA((_R,)),
            pltpu.SemaphoreType.DMA((_R,)),
            pltpu.SemaphoreType.DMA((_R,)),
        ],
        compiler_params=pltpu.CompilerParams(use_tc_tiling_on_sc=False),
    )(pe, idx1d)
    return out.reshape(BATCH, SEQ_LEN, D_MODEL)


def kernel(positions, pe):
    return _temporal_embedding(positions.astype(jnp.int32), pe)
